# baseline (device time: 26030 ns/iter reference)
import jax
import jax.numpy as jnp
from jax import lax
from jax.experimental import pallas as pl
from jax.experimental.pallas import tpu as pltpu

N_DEV = 16
WIRE_DTYPE = jnp.bfloat16
HALF_M = 512
Q = HALF_M // 2
E = HALF_M // 4
H = HALF_M // 8

ORDERS = ((0, 1, 2, 3), (1, 0, 3, 2))

(S0SA, S0SB, S0K, S1A, S1B, S2A, S2B, S3A, S3B, S4A, S4B,
 S5EA, S5EB, S5RA, S5RB) = range(15)


def kernel(x):
    m_per, n = x.shape

    def body(x_ref, out_ref, xb, agbuf, rb0s, rb0k, rb1, rb2, rb3,
             sb1, sb2, sb3, send_sems, recv_sems):
        d = lax.axis_index("i")
        z = d // 4
        p = lax.rem(d, 4)
        my_x = (p ^ (p >> 1)) & 1
        my_y = p >> 1

        partners = [
            4 * z + (p ^ 1),
            4 * z + (3 - p),
            4 * (z ^ 1) + p,
            4 * (z ^ 2) + p,
        ]
        bits = [my_x, my_y, z & 1, (z >> 1) & 1]

        send0, keep0, send1, keep1, b1s = [], [], [], [], []
        for k in range(2):
            D = ORDERS[k]
            b0, b1 = bits[D[0]], bits[D[1]]
            base = k * HALF_M
            send0.append(base + (1 - b0) * Q)
            keep0.append(base + b0 * Q)
            send1.append(keep0[k] + (1 - b1) * E)
            keep1.append(keep0[k] + b1 * E)
            b1s.append(b1)

        def make(k, slot, dim, src, dst):
            return pltpu.make_async_remote_copy(
                src_ref=src,
                dst_ref=dst,
                send_sem=send_sems.at[k, slot],
                recv_sem=recv_sems.at[k, slot],
                device_id=(partners[dim],),
                device_id_type=pl.DeviceIdType.MESH,
            )

        barrier_sem = pltpu.get_barrier_semaphore()
        for dim in range(4):
            pl.semaphore_signal(
                barrier_sem, inc=1,
                device_id=(partners[dim],),
                device_id_type=pl.DeviceIdType.MESH,
            )
        for k in range(2):
            xb[pl.ds(send0[k], Q), :] = (
                x_ref[pl.ds(send0[k], Q), :].astype(WIRE_DTYPE)
            )
        pl.semaphore_wait(barrier_sem, 4)

        r0s = [[None, None], [None, None]]
        r0k = []
        for h in range(2):
            for k in range(2):
                r = make(k, S0SA + h, ORDERS[k][0],
                         xb.at[pl.ds(send0[k] + (1 - b1s[k]) * E + h * H, H), :],
                         rb0s.at[k, pl.ds(h * H, H), :])
                r.start()
                r0s[k][h] = r
        for k in range(2):
            r = make(k, S0K, ORDERS[k][0],
                     xb.at[pl.ds(send0[k] + b1s[k] * E, E), :],
                     rb0k.at[k])
            r.start()
            r0k.append(r)

        r1 = [[None, None], [None, None]]
        for h in range(2):
            for k in range(2):
                r0s[k][h].wait_recv()
                sb1[k, pl.ds(h * H, H), :] = (
                    x_ref[pl.ds(send1[k] + h * H, H), :]
                    + rb0s[k, pl.ds(h * H, H), :].astype(jnp.float32)
                ).astype(WIRE_DTYPE)
                r = make(k, S1A + h, ORDERS[k][1],
                         sb1.at[k, pl.ds(h * H, H), :],
                         rb1.at[k, pl.ds(h * H, H), :])
                r.start()
                r1[k][h] = r
        for k in range(2):
            r0k[k].wait_recv()
            out_ref[pl.ds(keep1[k], E), :] = (
                x_ref[pl.ds(keep1[k], E), :]
                + rb0k[k, :, :].astype(jnp.float32)
            )

        def kh(k, h):
            return keep1[k] + h * H

        r2 = [[None, None], [None, None]]
        for h in range(2):
            for k in (1, 0):
                r1[k][h].wait_recv()
                acc = (
                    out_ref[pl.ds(kh(k, h), H), :]
                    + rb1[k, pl.ds(h * H, H), :].astype(jnp.float32)
                )
                out_ref[pl.ds(kh(k, h), H), :] = acc
                sb2[k, pl.ds(h * H, H), :] = acc.astype(WIRE_DTYPE)
                r = make(k, S2A + h, ORDERS[k][2],
                         sb2.at[k, pl.ds(h * H, H), :],
                         rb2.at[k, pl.ds(h * H, H), :])
                r.start()
                r2[k][h] = r

        r3 = [[None, None], [None, None]]
        for h in range(2):
            for k in range(2):
                r2[k][h].wait_recv()
                acc = (
                    out_ref[pl.ds(kh(k, h), H), :]
                    + rb2[k, pl.ds(h * H, H), :].astype(jnp.float32)
                )
                out_ref[pl.ds(kh(k, h), H), :] = acc
                sb3[k, pl.ds(h * H, H), :] = acc.astype(WIRE_DTYPE)
                r = make(k, S3A + h, ORDERS[k][3],
                         sb3.at[k, pl.ds(h * H, H), :],
                         rb3.at[k, pl.ds(h * H, H), :])
                r.start()
                r3[k][h] = r

        r4 = [[None, None], [None, None]]
        r5e = [[None, None], [None, None]]
        for h in range(2):
            for k in (1, 0):
                r3[k][h].wait_recv()
                acc = (
                    out_ref[pl.ds(kh(k, h), H), :]
                    + rb3[k, pl.ds(h * H, H), :].astype(jnp.float32)
                )
                out_ref[pl.ds(kh(k, h), H), :] = acc
                agbuf[pl.ds(kh(k, h), H), :] = acc.astype(WIRE_DTYPE)
                r = make(k, S4A + h, ORDERS[k][1],
                         agbuf.at[pl.ds(kh(k, h), H), :],
                         agbuf.at[pl.ds(kh(k, h), H), :])
                r.start()
                r4[k][h] = r
                r = make(k, S5EA + h, ORDERS[k][0],
                         agbuf.at[pl.ds(kh(k, h), H), :],
                         agbuf.at[pl.ds(kh(k, h), H), :])
                r.start()
                r5e[k][h] = r

        r5r = [[None, None], [None, None]]
        for h in range(2):
            for k in (1, 0):
                r4[k][h].wait_recv()
                o = send1[k] + h * H
                r = make(k, S5RA + h, ORDERS[k][0],
                         agbuf.at[pl.ds(o, H), :],
                         agbuf.at[pl.ds(o, H), :])
                r.start()
                r5r[k][h] = r
        for k in range(2):
            out_ref[pl.ds(send1[k], E), :] = (
                agbuf[pl.ds(send1[k], E), :].astype(jnp.float32)
            )
        for h in range(2):
            for k in (1, 0):
                r5e[k][h].wait_recv()
                o = send0[k] + b1s[k] * E + h * H
                out_ref[pl.ds(o, H), :] = (
                    agbuf[pl.ds(o, H), :].astype(jnp.float32)
                )
        for h in range(2):
            for k in (1, 0):
                r5r[k][h].wait_recv()
                o = send0[k] + (1 - b1s[k]) * E + h * H
                out_ref[pl.ds(o, H), :] = (
                    agbuf[pl.ds(o, H), :].astype(jnp.float32)
                )

        for k in range(2):
            r0k[k].wait_send()
            for h in range(2):
                for r in (r0s, r1, r2, r3, r4, r5e, r5r):
                    r[k][h].wait_send()

    return pl.pallas_call(
        body,
        out_shape=jax.ShapeDtypeStruct((m_per, n), x.dtype),
        in_specs=[pl.BlockSpec(memory_space=pltpu.VMEM)],
        out_specs=pl.BlockSpec(memory_space=pltpu.VMEM),
        scratch_shapes=[
            pltpu.VMEM((m_per, n), WIRE_DTYPE),
            pltpu.VMEM((m_per, n), WIRE_DTYPE),
            pltpu.VMEM((2, E, n), WIRE_DTYPE),
            pltpu.VMEM((2, E, n), WIRE_DTYPE),
            pltpu.VMEM((2, E, n), WIRE_DTYPE),
            pltpu.VMEM((2, E, n), WIRE_DTYPE),
            pltpu.VMEM((2, E, n), WIRE_DTYPE),
            pltpu.VMEM((2, E, n), WIRE_DTYPE),
            pltpu.VMEM((2, E, n), WIRE_DTYPE),
            pltpu.VMEM((2, E, n), WIRE_DTYPE),
            pltpu.SemaphoreType.DMA((2, 15)),
            pltpu.SemaphoreType.DMA((2, 15)),
        ],
        compiler_params=pltpu.CompilerParams(collective_id=0),
    )(x)
